# baseline (speedup 1.0000x reference)
import functools

import jax
import jax.numpy as jnp
from jax import lax
from jax.experimental import pallas as pl
from jax.experimental.pallas import tpu as pltpu
from jax.experimental.pallas import tpu_sc as plsc

N = 10000
E = 320000
D = 128

NC = 2
NS = 16
NW = NC * NS
GP = 128
RPT = 80
EP = NW * RPT * GP
RT = EP // GP
GPC = RT // NS
PADROWS = 240
NPAD = N + PADROWS
ROWS_PER_TILE = NPAD // NS
CH = 40


def _deg_body(srcp, dstp, zeros_nd, ones_g, out, cidx, ones_v, acc, sa, sb):
    c = lax.axis_index("c")
    s = lax.axis_index("s")

    pltpu.sync_copy(ones_g, ones_v)
    row0 = s * ROWS_PER_TILE
    pltpu.sync_copy(zeros_nd.at[pl.ds(row0, ROWS_PER_TILE)],
                    acc.at[pl.ds(row0, ROWS_PER_TILE)])
    plsc.subcore_barrier()

    def run(idxp):
        for chunk in range(GPC // CH):
            base = s * GPC + chunk * CH
            pltpu.sync_copy(idxp.at[pl.ds(base, CH)], cidx)

            def step(jj, carry):
                j = 2 * jj

                @pl.when(jj > 1)
                def _():
                    pltpu.make_async_copy(
                        ones_v, acc.at[cidx.at[0]], sa).wait()
                    pltpu.make_async_copy(
                        ones_v, acc.at[cidx.at[0]], sb).wait()

                pltpu.async_copy(ones_v, acc.at[cidx.at[j]], sa, add=True)
                pltpu.async_copy(ones_v, acc.at[cidx.at[j + 1]], sb, add=True)
                return carry

            lax.fori_loop(0, CH // 2, step, 0)
            for _ in range(2):
                pltpu.make_async_copy(ones_v, acc.at[cidx.at[0]], sa).wait()
                pltpu.make_async_copy(ones_v, acc.at[cidx.at[0]], sb).wait()

    @pl.when(c == 0)
    def _():
        run(srcp)

    @pl.when(c == 1)
    def _():
        run(dstp)

    plsc.subcore_barrier()
    pltpu.sync_copy(acc.at[pl.ds(row0, ROWS_PER_TILE)],
                    out.at[c, pl.ds(row0, ROWS_PER_TILE)])


_deg_kernel = functools.partial(
    pl.kernel,
    out_type=jax.ShapeDtypeStruct((NC, NPAD, D), jnp.float32),
    mesh=plsc.VectorSubcoreMesh(core_axis_name="c", subcore_axis_name="s"),
    scratch_types=[
        pltpu.VMEM((CH, GP), jnp.int32),
        pltpu.VMEM((GP, D), jnp.float32),
        pltpu.VMEM_SHARED((NPAD, D), jnp.float32),
        pltpu.SemaphoreType.DMA,
        pltpu.SemaphoreType.DMA,
    ],
)(_deg_body)


def _agg_body(hpad, srcp, dstp, zeros_nd, out, sidx, didx, rows_a, rows_b,
              acc, ga, gb, sa, sb):
    c = lax.axis_index("c")
    s = lax.axis_index("s")
    g = c * NS + s

    pltpu.sync_copy(srcp.at[pl.ds(g * RPT, CH)], sidx)
    pltpu.sync_copy(dstp.at[pl.ds(g * RPT, CH)], didx)
    pltpu.async_copy(hpad.at[sidx.at[0]], rows_a, ga)
    pltpu.async_copy(hpad.at[sidx.at[1]], rows_b, gb)

    row0 = s * ROWS_PER_TILE
    pltpu.sync_copy(zeros_nd.at[pl.ds(row0, ROWS_PER_TILE)],
                    acc.at[pl.ds(row0, ROWS_PER_TILE)])
    plsc.subcore_barrier()

    for chunk in range(RPT // CH):
        if chunk:
            base = g * RPT + chunk * CH
            pltpu.sync_copy(srcp.at[pl.ds(base, CH)], sidx)
            pltpu.sync_copy(dstp.at[pl.ds(base, CH)], didx)
            pltpu.async_copy(hpad.at[sidx.at[0]], rows_a, ga)
            pltpu.async_copy(hpad.at[sidx.at[1]], rows_b, gb)

        def step(jj, carry):
            j = 2 * jj
            pltpu.make_async_copy(hpad.at[sidx.at[j]], rows_a, ga).wait()
            pltpu.async_copy(rows_a, acc.at[didx.at[j]], sa, add=True)
            pltpu.make_async_copy(hpad.at[sidx.at[j + 1]], rows_b, gb).wait()
            pltpu.async_copy(rows_b, acc.at[didx.at[j + 1]], sb, add=True)

            @pl.when(jj < CH // 2 - 1)
            def _():
                pltpu.make_async_copy(rows_a, acc.at[didx.at[0]], sa).wait()
                pltpu.async_copy(hpad.at[sidx.at[j + 2]], rows_a, ga)
                pltpu.make_async_copy(rows_b, acc.at[didx.at[0]], sb).wait()
                pltpu.async_copy(hpad.at[sidx.at[j + 3]], rows_b, gb)

            return carry

        lax.fori_loop(0, CH // 2, step, 0)
        pltpu.make_async_copy(rows_a, acc.at[didx.at[0]], sa).wait()
        pltpu.make_async_copy(rows_b, acc.at[didx.at[0]], sb).wait()

    plsc.subcore_barrier()
    pltpu.sync_copy(acc.at[pl.ds(row0, ROWS_PER_TILE)],
                    out.at[c, pl.ds(row0, ROWS_PER_TILE)])


_agg_kernel = functools.partial(
    pl.kernel,
    out_type=jax.ShapeDtypeStruct((NC, NPAD, D), jnp.float32),
    mesh=plsc.VectorSubcoreMesh(core_axis_name="c", subcore_axis_name="s"),
    scratch_types=[
        pltpu.VMEM((CH, GP), jnp.int32),
        pltpu.VMEM((CH, GP), jnp.int32),
        pltpu.VMEM((GP, D), jnp.float32),
        pltpu.VMEM((GP, D), jnp.float32),
        pltpu.VMEM_SHARED((NPAD, D), jnp.float32),
        pltpu.SemaphoreType.DMA,
        pltpu.SemaphoreType.DMA,
        pltpu.SemaphoreType.DMA,
        pltpu.SemaphoreType.DMA,
    ],
)(_agg_body)


R = 1000


def _prep_body(feats, dego, degi, h0s, nsrc, ndst):
    do = dego[0, :, 0:1]
    di = degi[0, :, 0:1]
    ns = jax.lax.rsqrt(jnp.where(do > 0, do, 1.0))
    nd = jax.lax.rsqrt(jnp.where(di > 0, di, 1.0))
    nsrc[...] = ns
    ndst[...] = nd
    h0s[...] = feats[...] * ns


def _prep_call(feats, deg):
    return pl.pallas_call(
        _prep_body,
        grid=(N // R,),
        in_specs=[
            pl.BlockSpec((R, D), lambda i: (i, 0)),
            pl.BlockSpec((1, R, D), lambda i: (0, i, 0)),
            pl.BlockSpec((1, R, D), lambda i: (1, i, 0)),
        ],
        out_specs=[
            pl.BlockSpec((R, D), lambda i: (i, 0)),
            pl.BlockSpec((R, 1), lambda i: (i, 0)),
            pl.BlockSpec((R, 1), lambda i: (i, 0)),
        ],
        out_shape=[
            jax.ShapeDtypeStruct((NPAD, D), jnp.float32),
            jax.ShapeDtypeStruct((N, 1), jnp.float32),
            jax.ShapeDtypeStruct((N, 1), jnp.float32),
        ],
    )(feats, deg, deg)


def _layer1_body(part, ndst, nsrc, w, b, h_out, hs_out):
    agg = (part[0] + part[1]) * ndst[...]
    y = jnp.dot(agg, w[...], preferred_element_type=jnp.float32) + b[...]
    y = jnp.maximum(y, 0.0)
    h_out[...] = y
    hs_out[...] = y * nsrc[...]


def _layer2_body(part, ndst, nsrc, w, b, h_out):
    agg = (part[0] + part[1]) * ndst[...]
    h_out[...] = jnp.dot(agg, w[...], preferred_element_type=jnp.float32) + b[...]


def _layer_call(part, ndst, nsrc, w, b, first):
    if first:
        body = _layer1_body
        out_specs = [pl.BlockSpec((R, D), lambda i: (i, 0)),
                     pl.BlockSpec((R, D), lambda i: (i, 0))]
        out_shape = [jax.ShapeDtypeStruct((N, D), jnp.float32),
                     jax.ShapeDtypeStruct((NPAD, D), jnp.float32)]
    else:
        body = _layer2_body
        out_specs = pl.BlockSpec((R, D), lambda i: (i, 0))
        out_shape = jax.ShapeDtypeStruct((N, D), jnp.float32)
    return pl.pallas_call(
        body,
        grid=(N // R,),
        in_specs=[
            pl.BlockSpec((NC, R, D), lambda i: (0, i, 0)),
            pl.BlockSpec((R, 1), lambda i: (i, 0)),
            pl.BlockSpec((R, 1), lambda i: (i, 0)),
            pl.BlockSpec((D, D), lambda i: (0, 0)),
            pl.BlockSpec((1, D), lambda i: (0, 0)),
        ],
        out_specs=out_specs,
        out_shape=out_shape,
    )(part, ndst, nsrc, w, b)


def kernel(feats, edge_index, W1, b1, W2, b2):
    src = edge_index[0]
    dst = edge_index[1]
    pad = N + (jnp.arange(EP - E, dtype=jnp.int32) % PADROWS)
    srcp = jnp.concatenate([src, pad]).reshape(RT, GP)
    dstp = jnp.concatenate([dst, pad]).reshape(RT, GP)

    zeros_nd = jnp.zeros((NPAD, D), jnp.float32)
    ones_g = jnp.ones((GP, D), jnp.float32)

    deg = _deg_kernel(srcp, dstp, zeros_nd, ones_g)

    h0s, nsrc, ndst = _prep_call(feats, deg)

    part1 = _agg_kernel(h0s, srcp, dstp, zeros_nd)
    h1, h1s = _layer_call(part1, ndst, nsrc, W1, b1.reshape(1, D), True)

    part2 = _agg_kernel(h1s, srcp, dstp, zeros_nd)
    h2 = _layer_call(part2, ndst, nsrc, W2, b2.reshape(1, D), False)

    return (h1, h2)


# speedup vs baseline: 1.0557x; 1.0557x over previous
import functools

import jax
import jax.numpy as jnp
from jax import lax
from jax.experimental import pallas as pl
from jax.experimental.pallas import tpu as pltpu
from jax.experimental.pallas import tpu_sc as plsc

N = 10000
E = 320000
D = 128

NC = 2
NS = 16
NW = NC * NS
GP = 128
RPT = 80
EP = NW * RPT * GP
RT = EP // GP
GPC = RT // NS
PADROWS = 240
NPAD = N + PADROWS
ROWS_PER_TILE = NPAD // NS
CH = 40
HB = NPAD * 8


def _deg_body(srcp, dstp, out, cidx, tmp, hist, sem):
    c = lax.axis_index("c")
    s = lax.axis_index("s")
    g = c * NS + s
    lane = lax.iota(jnp.int32, 16)
    lo = lane < 8
    sub = lane & 7

    def phase(arr, ph):
        pltpu.sync_copy(arr.at[pl.ds(g * RPT, RPT)], cidx)

        def zstep(z, carry):
            hist[pl.ds(z * 16, 16)] = jnp.zeros((16,), jnp.float32)
            return carry

        lax.fori_loop(0, HB // 16, zstep, 0)

        def step(k, carry):
            v = cidx[k >> 3, pl.ds((k & 7) * 16, 16)]
            tmp[pl.ds(0, 16)] = v
            tmp[pl.ds(16, 16)] = v
            rot = tmp[pl.ds(8, 16)]
            eq = v == rot
            val = jnp.where(lo & eq, 2.0, 1.0).astype(jnp.float32)
            msk = lo | jnp.logical_not(eq)
            plsc.addupdate_scatter(hist, [v * 8 + sub], val, mask=msk)
            return carry

        lax.fori_loop(0, RPT * 8, step, 0)
        pltpu.sync_copy(hist, out.at[g, ph])

    phase(srcp, 0)
    phase(dstp, 1)


_deg_kernel = functools.partial(
    pl.kernel,
    compiler_params=pltpu.CompilerParams(needs_layout_passes=False),
    out_type=jax.ShapeDtypeStruct((NW, 2, HB), jnp.float32),
    mesh=plsc.VectorSubcoreMesh(core_axis_name="c", subcore_axis_name="s"),
    scratch_types=[
        pltpu.VMEM((RPT, GP), jnp.int32),
        pltpu.VMEM((32,), jnp.int32),
        pltpu.VMEM((HB,), jnp.float32),
        pltpu.SemaphoreType.DMA,
    ],
)(_deg_body)


def _agg_body(hpad, srcp, dstp, zeros_nd, out, sidx, didx, rows_a, rows_b,
              acc, ga, gb, sa, sb):
    c = lax.axis_index("c")
    s = lax.axis_index("s")
    g = c * NS + s

    pltpu.sync_copy(srcp.at[pl.ds(g * RPT, CH)], sidx)
    pltpu.sync_copy(dstp.at[pl.ds(g * RPT, CH)], didx)
    pltpu.async_copy(hpad.at[sidx.at[0]], rows_a, ga)
    pltpu.async_copy(hpad.at[sidx.at[1]], rows_b, gb)

    row0 = s * ROWS_PER_TILE
    pltpu.sync_copy(zeros_nd.at[pl.ds(row0, ROWS_PER_TILE)],
                    acc.at[pl.ds(row0, ROWS_PER_TILE)])
    plsc.subcore_barrier()

    for chunk in range(RPT // CH):
        if chunk:
            base = g * RPT + chunk * CH
            pltpu.sync_copy(srcp.at[pl.ds(base, CH)], sidx)
            pltpu.sync_copy(dstp.at[pl.ds(base, CH)], didx)
            pltpu.async_copy(hpad.at[sidx.at[0]], rows_a, ga)
            pltpu.async_copy(hpad.at[sidx.at[1]], rows_b, gb)

        def step(jj, carry):
            j = 2 * jj
            pltpu.make_async_copy(hpad.at[sidx.at[j]], rows_a, ga).wait()
            pltpu.async_copy(rows_a, acc.at[didx.at[j]], sa, add=True)
            pltpu.make_async_copy(hpad.at[sidx.at[j + 1]], rows_b, gb).wait()
            pltpu.async_copy(rows_b, acc.at[didx.at[j + 1]], sb, add=True)

            @pl.when(jj < CH // 2 - 1)
            def _():
                pltpu.make_async_copy(rows_a, acc.at[didx.at[0]], sa).wait()
                pltpu.async_copy(hpad.at[sidx.at[j + 2]], rows_a, ga)
                pltpu.make_async_copy(rows_b, acc.at[didx.at[0]], sb).wait()
                pltpu.async_copy(hpad.at[sidx.at[j + 3]], rows_b, gb)

            return carry

        lax.fori_loop(0, CH // 2, step, 0)
        pltpu.make_async_copy(rows_a, acc.at[didx.at[0]], sa).wait()
        pltpu.make_async_copy(rows_b, acc.at[didx.at[0]], sb).wait()

    plsc.subcore_barrier()
    pltpu.sync_copy(acc.at[pl.ds(row0, ROWS_PER_TILE)],
                    out.at[c, pl.ds(row0, ROWS_PER_TILE)])


_agg_kernel = functools.partial(
    pl.kernel,
    out_type=jax.ShapeDtypeStruct((NC, NPAD, D), jnp.float32),
    mesh=plsc.VectorSubcoreMesh(core_axis_name="c", subcore_axis_name="s"),
    scratch_types=[
        pltpu.VMEM((CH, GP), jnp.int32),
        pltpu.VMEM((CH, GP), jnp.int32),
        pltpu.VMEM((GP, D), jnp.float32),
        pltpu.VMEM((GP, D), jnp.float32),
        pltpu.VMEM_SHARED((NPAD, D), jnp.float32),
        pltpu.SemaphoreType.DMA,
        pltpu.SemaphoreType.DMA,
        pltpu.SemaphoreType.DMA,
        pltpu.SemaphoreType.DMA,
    ],
)(_agg_body)


R = 1000


R2 = 2048


def _prep_body(feats, dego, degi, h0s, nsrc, ndst):
    do = jnp.sum(dego[...], axis=(0, 1)).reshape(R2 // 16, 16, 8)
    do = jnp.sum(do, axis=-1).reshape(R2, 1)
    di = jnp.sum(degi[...], axis=(0, 1)).reshape(R2 // 16, 16, 8)
    di = jnp.sum(di, axis=-1).reshape(R2, 1)
    ns = jax.lax.rsqrt(jnp.where(do > 0, do, 1.0))
    nd = jax.lax.rsqrt(jnp.where(di > 0, di, 1.0))
    nsrc[...] = ns
    ndst[...] = nd
    h0s[...] = feats[...] * ns


def _prep_call(feats, deg):
    return pl.pallas_call(
        _prep_body,
        grid=(NPAD // R2,),
        in_specs=[
            pl.BlockSpec((R2, D), lambda i: (i, 0)),
            pl.BlockSpec((NW, 1, R2 // 16, 128), lambda i: (0, 0, i, 0)),
            pl.BlockSpec((NW, 1, R2 // 16, 128), lambda i: (0, 1, i, 0)),
        ],
        out_specs=[
            pl.BlockSpec((R2, D), lambda i: (i, 0)),
            pl.BlockSpec((R2, 1), lambda i: (i, 0)),
            pl.BlockSpec((R2, 1), lambda i: (i, 0)),
        ],
        out_shape=[
            jax.ShapeDtypeStruct((NPAD, D), jnp.float32),
            jax.ShapeDtypeStruct((NPAD, 1), jnp.float32),
            jax.ShapeDtypeStruct((NPAD, 1), jnp.float32),
        ],
    )(feats, deg, deg)


def _layer1_body(part, ndst, nsrc, w, b, h_out, hs_out):
    agg = (part[0] + part[1]) * ndst[...]
    y = jnp.dot(agg, w[...], preferred_element_type=jnp.float32) + b[...]
    y = jnp.maximum(y, 0.0)
    h_out[...] = y
    hs_out[...] = y * nsrc[...]


def _layer2_body(part, ndst, nsrc, w, b, h_out):
    agg = (part[0] + part[1]) * ndst[...]
    h_out[...] = jnp.dot(agg, w[...], preferred_element_type=jnp.float32) + b[...]


def _layer_call(part, ndst, nsrc, w, b, first):
    if first:
        body = _layer1_body
        out_specs = [pl.BlockSpec((R, D), lambda i: (i, 0)),
                     pl.BlockSpec((R, D), lambda i: (i, 0))]
        out_shape = [jax.ShapeDtypeStruct((N, D), jnp.float32),
                     jax.ShapeDtypeStruct((NPAD, D), jnp.float32)]
    else:
        body = _layer2_body
        out_specs = pl.BlockSpec((R, D), lambda i: (i, 0))
        out_shape = jax.ShapeDtypeStruct((N, D), jnp.float32)
    return pl.pallas_call(
        body,
        grid=(N // R,),
        in_specs=[
            pl.BlockSpec((NC, R, D), lambda i: (0, i, 0)),
            pl.BlockSpec((R, 1), lambda i: (i, 0)),
            pl.BlockSpec((R, 1), lambda i: (i, 0)),
            pl.BlockSpec((D, D), lambda i: (0, 0)),
            pl.BlockSpec((1, D), lambda i: (0, 0)),
        ],
        out_specs=out_specs,
        out_shape=out_shape,
    )(part, ndst, nsrc, w, b)


def kernel(feats, edge_index, W1, b1, W2, b2):
    src = edge_index[0]
    dst = edge_index[1]
    pad = N + (jnp.arange(EP - E, dtype=jnp.int32) % PADROWS)
    srcp = jnp.concatenate([src, pad]).reshape(RT, GP)
    dstp = jnp.concatenate([dst, pad]).reshape(RT, GP)

    zeros_nd = jnp.zeros((NPAD, D), jnp.float32)

    deg = _deg_kernel(srcp, dstp).reshape(NW, 2, NPAD // 16, 128)

    h0s, nsrc, ndst = _prep_call(jnp.pad(feats, ((0, PADROWS), (0, 0))), deg)

    part1 = _agg_kernel(h0s, srcp, dstp, zeros_nd)
    h1, h1s = _layer_call(part1, ndst, nsrc, W1, b1.reshape(1, D), True)

    part2 = _agg_kernel(h1s, srcp, dstp, zeros_nd)
    h2 = _layer_call(part2, ndst, nsrc, W2, b2.reshape(1, D), False)

    return (h1, h2)


# speedup vs baseline: 1.1481x; 1.0875x over previous
import functools

import jax
import jax.numpy as jnp
from jax import lax
from jax.experimental import pallas as pl
from jax.experimental.pallas import tpu as pltpu
from jax.experimental.pallas import tpu_sc as plsc

N = 10000
E = 320000
D = 128

NC = 2
NS = 16
NW = NC * NS
GP = 128
RPT = 80
EP = NW * RPT * GP
RT = EP // GP
GPC = RT // NS
PADROWS = 240
NPAD = N + PADROWS
ROWS_PER_TILE = NPAD // NS
CH = 40
HB = NPAD * 8


def _deg_body(srcp, dstp, out, cidx, tmp, hist, sem):
    c = lax.axis_index("c")
    s = lax.axis_index("s")
    g = c * NS + s
    lane = lax.iota(jnp.int32, 16)
    lo = lane < 8
    sub = lane & 7

    def phase(arr, ph):
        pltpu.sync_copy(arr.at[pl.ds(g * RPT, RPT)], cidx)

        zv = jnp.zeros((16,), jnp.float32)

        def zstep(z, carry):
            for u in range(8):
                hist[pl.ds(z * 128 + u * 16, 16)] = zv
            return carry

        lax.fori_loop(0, HB // 128, zstep, 0)

        def step(r, carry):
            for u in range(8):
                v = cidx[r, pl.ds(u * 16, 16)]
                tmp[pl.ds(0, 16)] = v
                tmp[pl.ds(16, 16)] = v
                rot = tmp[pl.ds(8, 16)]
                eq = v == rot
                val = jnp.where(lo & eq, 2.0, 1.0).astype(jnp.float32)
                msk = lo | jnp.logical_not(eq)
                plsc.addupdate_scatter(hist, [v * 8 + sub], val, mask=msk)
            return carry

        lax.fori_loop(0, RPT, step, 0)
        pltpu.sync_copy(hist, out.at[g, ph])

    phase(srcp, 0)
    phase(dstp, 1)


_deg_kernel = functools.partial(
    pl.kernel,
    compiler_params=pltpu.CompilerParams(needs_layout_passes=False),
    out_type=jax.ShapeDtypeStruct((NW, 2, HB), jnp.float32),
    mesh=plsc.VectorSubcoreMesh(core_axis_name="c", subcore_axis_name="s"),
    scratch_types=[
        pltpu.VMEM((RPT, GP), jnp.int32),
        pltpu.VMEM((32,), jnp.int32),
        pltpu.VMEM((HB,), jnp.float32),
        pltpu.SemaphoreType.DMA,
    ],
)(_deg_body)


def _agg_body(hpad, srcp, dstp, zeros_nd, out, sidx, didx, rows_a, rows_b,
              acc, ga, gb, sa, sb):
    c = lax.axis_index("c")
    s = lax.axis_index("s")
    g = c * NS + s

    pltpu.sync_copy(srcp.at[pl.ds(g * RPT, CH)], sidx)
    pltpu.sync_copy(dstp.at[pl.ds(g * RPT, CH)], didx)
    pltpu.async_copy(hpad.at[sidx.at[0]], rows_a, ga)
    pltpu.async_copy(hpad.at[sidx.at[1]], rows_b, gb)

    row0 = s * ROWS_PER_TILE
    pltpu.sync_copy(zeros_nd.at[pl.ds(row0, ROWS_PER_TILE)],
                    acc.at[pl.ds(row0, ROWS_PER_TILE)])
    plsc.subcore_barrier()

    for chunk in range(RPT // CH):
        if chunk:
            base = g * RPT + chunk * CH
            pltpu.sync_copy(srcp.at[pl.ds(base, CH)], sidx)
            pltpu.sync_copy(dstp.at[pl.ds(base, CH)], didx)
            pltpu.async_copy(hpad.at[sidx.at[0]], rows_a, ga)
            pltpu.async_copy(hpad.at[sidx.at[1]], rows_b, gb)

        def step(jj, carry):
            j = 2 * jj
            pltpu.make_async_copy(hpad.at[sidx.at[j]], rows_a, ga).wait()
            pltpu.async_copy(rows_a, acc.at[didx.at[j]], sa, add=True)
            pltpu.make_async_copy(hpad.at[sidx.at[j + 1]], rows_b, gb).wait()
            pltpu.async_copy(rows_b, acc.at[didx.at[j + 1]], sb, add=True)

            @pl.when(jj < CH // 2 - 1)
            def _():
                pltpu.make_async_copy(rows_a, acc.at[didx.at[0]], sa).wait()
                pltpu.async_copy(hpad.at[sidx.at[j + 2]], rows_a, ga)
                pltpu.make_async_copy(rows_b, acc.at[didx.at[0]], sb).wait()
                pltpu.async_copy(hpad.at[sidx.at[j + 3]], rows_b, gb)

            return carry

        lax.fori_loop(0, CH // 2, step, 0)
        pltpu.make_async_copy(rows_a, acc.at[didx.at[0]], sa).wait()
        pltpu.make_async_copy(rows_b, acc.at[didx.at[0]], sb).wait()

    plsc.subcore_barrier()
    pltpu.sync_copy(acc.at[pl.ds(row0, ROWS_PER_TILE)],
                    out.at[c, pl.ds(row0, ROWS_PER_TILE)])


_agg_kernel = functools.partial(
    pl.kernel,
    out_type=jax.ShapeDtypeStruct((NC, NPAD, D), jnp.float32),
    mesh=plsc.VectorSubcoreMesh(core_axis_name="c", subcore_axis_name="s"),
    scratch_types=[
        pltpu.VMEM((CH, GP), jnp.int32),
        pltpu.VMEM((CH, GP), jnp.int32),
        pltpu.VMEM((GP, D), jnp.float32),
        pltpu.VMEM((GP, D), jnp.float32),
        pltpu.VMEM_SHARED((NPAD, D), jnp.float32),
        pltpu.SemaphoreType.DMA,
        pltpu.SemaphoreType.DMA,
        pltpu.SemaphoreType.DMA,
        pltpu.SemaphoreType.DMA,
    ],
)(_agg_body)


R = 1000


R2 = 2048


def _prep_body(feats, dego, degi, h0s, nsrc, ndst):
    do = jnp.sum(dego[...], axis=(0, 1)).reshape(R2 // 16, 16, 8)
    do = jnp.sum(do, axis=-1).reshape(R2, 1)
    di = jnp.sum(degi[...], axis=(0, 1)).reshape(R2 // 16, 16, 8)
    di = jnp.sum(di, axis=-1).reshape(R2, 1)
    ns = jax.lax.rsqrt(jnp.where(do > 0, do, 1.0))
    nd = jax.lax.rsqrt(jnp.where(di > 0, di, 1.0))
    nsrc[...] = ns
    ndst[...] = nd
    h0s[...] = feats[...] * ns


def _prep_call(feats, deg):
    return pl.pallas_call(
        _prep_body,
        grid=(NPAD // R2,),
        in_specs=[
            pl.BlockSpec((R2, D), lambda i: (i, 0)),
            pl.BlockSpec((NW, 1, R2 // 16, 128), lambda i: (0, 0, i, 0)),
            pl.BlockSpec((NW, 1, R2 // 16, 128), lambda i: (0, 1, i, 0)),
        ],
        out_specs=[
            pl.BlockSpec((R2, D), lambda i: (i, 0)),
            pl.BlockSpec((R2, 1), lambda i: (i, 0)),
            pl.BlockSpec((R2, 1), lambda i: (i, 0)),
        ],
        out_shape=[
            jax.ShapeDtypeStruct((NPAD, D), jnp.float32),
            jax.ShapeDtypeStruct((NPAD, 1), jnp.float32),
            jax.ShapeDtypeStruct((NPAD, 1), jnp.float32),
        ],
    )(feats, deg, deg)


def _layer1_body(part, ndst, nsrc, w, b, h_out, hs_out):
    agg = (part[0] + part[1]) * ndst[...]
    y = jnp.dot(agg, w[...], preferred_element_type=jnp.float32) + b[...]
    y = jnp.maximum(y, 0.0)
    h_out[...] = y
    hs_out[...] = y * nsrc[...]


def _layer2_body(part, ndst, nsrc, w, b, h_out):
    agg = (part[0] + part[1]) * ndst[...]
    h_out[...] = jnp.dot(agg, w[...], preferred_element_type=jnp.float32) + b[...]


def _layer_call(part, ndst, nsrc, w, b, first):
    if first:
        body = _layer1_body
        out_specs = [pl.BlockSpec((R, D), lambda i: (i, 0)),
                     pl.BlockSpec((R, D), lambda i: (i, 0))]
        out_shape = [jax.ShapeDtypeStruct((N, D), jnp.float32),
                     jax.ShapeDtypeStruct((NPAD, D), jnp.float32)]
    else:
        body = _layer2_body
        out_specs = pl.BlockSpec((R, D), lambda i: (i, 0))
        out_shape = jax.ShapeDtypeStruct((N, D), jnp.float32)
    return pl.pallas_call(
        body,
        grid=(N // R,),
        in_specs=[
            pl.BlockSpec((NC, R, D), lambda i: (0, i, 0)),
            pl.BlockSpec((R, 1), lambda i: (i, 0)),
            pl.BlockSpec((R, 1), lambda i: (i, 0)),
            pl.BlockSpec((D, D), lambda i: (0, 0)),
            pl.BlockSpec((1, D), lambda i: (0, 0)),
        ],
        out_specs=out_specs,
        out_shape=out_shape,
    )(part, ndst, nsrc, w, b)


def kernel(feats, edge_index, W1, b1, W2, b2):
    src = edge_index[0]
    dst = edge_index[1]
    pad = N + (jnp.arange(EP - E, dtype=jnp.int32) % PADROWS)
    srcp = jnp.concatenate([src, pad]).reshape(RT, GP)
    dstp = jnp.concatenate([dst, pad]).reshape(RT, GP)

    zeros_nd = jnp.zeros((NPAD, D), jnp.float32)

    deg = _deg_kernel(srcp, dstp).reshape(NW, 2, NPAD // 16, 128)

    h0s, nsrc, ndst = _prep_call(jnp.pad(feats, ((0, PADROWS), (0, 0))), deg)

    part1 = _agg_kernel(h0s, srcp, dstp, zeros_nd)
    h1, h1s = _layer_call(part1, ndst, nsrc, W1, b1.reshape(1, D), True)

    part2 = _agg_kernel(h1s, srcp, dstp, zeros_nd)
    h2 = _layer_call(part2, ndst, nsrc, W2, b2.reshape(1, D), False)

    return (h1, h2)
